# baseline (device time: 38764 ns/iter reference)
import os

import jax
import jax.numpy as jnp
from jax import lax
from jax.experimental import pallas as pl
from jax.experimental.pallas import tpu as pltpu

Z = 4
M = 1024
MQ = M // 4
C = 4
SH = MQ // C
N = 2048
NP = N // Z

_ABLATE = os.environ.get("KERNEL_ABLATE", "")


def kernel(x):
    def body(x_ref, out_ref, comm_ref, zsend, zrecv, fsend, frecv, fbar):
        mx = lax.axis_index("x")
        my = lax.axis_index("y")
        mz = lax.axis_index("z")
        left = (mz + Z - 1) % Z
        right = (mz + 1) % Z
        q = my % 2
        ypart = my - q + (1 - q)
        ro = q * 512 + mx * MQ
        fan = (
            (1 - mx, my, mz),
            (mx, ypart, mz),
            (1 - mx, ypart, mz),
        )

        barrier_sem = pltpu.get_barrier_semaphore()
        pl.semaphore_signal(
            barrier_sem, inc=1,
            device_id=(mx, my, left), device_id_type=pl.DeviceIdType.MESH,
        )
        for did in fan:
            pl.semaphore_signal(
                fbar, inc=1,
                device_id=did, device_id_type=pl.DeviceIdType.MESH,
            )
        pl.semaphore_wait(barrier_sem, 1)

        def z_send(s, c):
            send_j = (mz + Z - 1 - s) % Z
            if s == 0:
                src = x_ref.at[
                    0, pl.ds(ro + c * SH, SH), pl.ds(send_j * NP, NP)
                ]
            else:
                src = comm_ref.at[s - 1, pl.ds(c * SH, SH)]
            rdma = pltpu.make_async_remote_copy(
                src_ref=src,
                dst_ref=comm_ref.at[s, pl.ds(c * SH, SH)],
                send_sem=zsend.at[s, c],
                recv_sem=zrecv.at[s, c],
                device_id=(mx, my, right),
                device_id_type=pl.DeviceIdType.MESH,
            )
            rdma.start()
            return rdma

        def swap_out(src_ref, row0, send_sem, recv_sem, target):
            rdma = pltpu.make_async_remote_copy(
                src_ref=src_ref,
                dst_ref=out_ref.at[pl.ds(row0, SH), :],
                send_sem=send_sem,
                recv_sem=recv_sem,
                device_id=target,
                device_id_type=pl.DeviceIdType.MESH,
            )
            rdma.start()
            return rdma

        pending = {}
        drains = []
        if _ABLATE == "barrier":
            out_ref[...] = x_ref[0, :, pl.ds(0, NP)]
            return
        P = 2
        for c in range(P):
            pending[(0, c)] = z_send(0, c)
        next0 = P

        for s in range(Z - 1):
            recv_j = (mz + Z - 2 - s) % Z
            for c in range(C):
                pending[(s, c)].wait_recv()
                comm_ref[s, pl.ds(c * SH, SH)] = comm_ref[
                    s, pl.ds(c * SH, SH)
                ] + x_ref[0, pl.ds(ro + c * SH, SH), pl.ds(recv_j * NP, NP)]
                if s < Z - 2:
                    pending[(s + 1, c)] = z_send(s + 1, c)
                    if next0 < C:
                        pending[(0, next0)] = z_send(0, next0)
                        next0 += 1
                else:
                    final = comm_ref.at[s, pl.ds(c * SH, SH)]
                    out_ref[pl.ds(ro + c * SH, SH), :] = comm_ref[
                        s, pl.ds(c * SH, SH)
                    ]
                    if _ABLATE != "zonly":
                        if c == 0:
                            pl.semaphore_wait(fbar, 3)
                        for k, target in enumerate(fan):
                            drains.append(
                                swap_out(final, ro + c * SH, fsend.at[k, c],
                                         frecv.at[k, c], target)
                            )

        for rdma in list(pending.values()) + drains:
            rdma.wait_send()
        for rdma in drains:
            rdma.wait_recv()

    return pl.pallas_call(
        body,
        out_shape=jax.ShapeDtypeStruct((M, NP), jnp.float32),
        in_specs=[pl.BlockSpec(memory_space=pltpu.VMEM)],
        out_specs=pl.BlockSpec(memory_space=pltpu.VMEM),
        scratch_shapes=[
            pltpu.VMEM((Z - 1, MQ, NP), jnp.float32),
            pltpu.SemaphoreType.DMA((Z - 1, C)),
            pltpu.SemaphoreType.DMA((Z - 1, C)),
            pltpu.SemaphoreType.DMA((3, C)),
            pltpu.SemaphoreType.DMA((3, C)),
            pltpu.SemaphoreType.REGULAR,
        ],
        compiler_params=pltpu.CompilerParams(collective_id=0),
    )(x)


# device time: 37746 ns/iter; 1.0270x vs baseline; 1.0270x over previous
import os

import jax
import jax.numpy as jnp
from jax import lax
from jax.experimental import pallas as pl
from jax.experimental.pallas import tpu as pltpu

Z = 4
M = 1024
MQ = M // 4
C = 4
SH = MQ // C
N = 2048
NP = N // Z

_ABLATE = os.environ.get("KERNEL_ABLATE", "")


def kernel(x):
    def body(x_ref, out_ref, comm_ref, zsend, zrecv, fsend, frecv, fbar):
        mx = lax.axis_index("x")
        my = lax.axis_index("y")
        mz = lax.axis_index("z")
        left = (mz + Z - 1) % Z
        right = (mz + 1) % Z
        q = my % 2
        ypart = my - q + (1 - q)
        ro = q * 512 + mx * MQ
        fan = (
            (1 - mx, my, mz),
            (mx, ypart, mz),
            (1 - mx, ypart, mz),
        )

        if _ABLATE == "launch":
            out_ref[...] = x_ref[0, :, pl.ds(0, NP)]
            return

        barrier_sem = pltpu.get_barrier_semaphore()
        pl.semaphore_signal(
            barrier_sem, inc=1,
            device_id=(mx, my, left), device_id_type=pl.DeviceIdType.MESH,
        )
        for did in fan:
            pl.semaphore_signal(
                fbar, inc=1,
                device_id=did, device_id_type=pl.DeviceIdType.MESH,
            )
        pl.semaphore_wait(barrier_sem, 1)

        def z_send(s, c):
            send_j = (mz + Z - 1 - s) % Z
            if s == 0:
                src = x_ref.at[
                    0, pl.ds(ro + c * SH, SH), pl.ds(send_j * NP, NP)
                ]
            else:
                src = comm_ref.at[s - 1, pl.ds(c * SH, SH)]
            rdma = pltpu.make_async_remote_copy(
                src_ref=src,
                dst_ref=comm_ref.at[s, pl.ds(c * SH, SH)],
                send_sem=zsend.at[s, c],
                recv_sem=zrecv.at[s, c],
                device_id=(mx, my, right),
                device_id_type=pl.DeviceIdType.MESH,
            )
            rdma.start()
            return rdma

        def swap_out(src_ref, row0, send_sem, recv_sem, target):
            rdma = pltpu.make_async_remote_copy(
                src_ref=src_ref,
                dst_ref=out_ref.at[pl.ds(row0, SH), :],
                send_sem=send_sem,
                recv_sem=recv_sem,
                device_id=target,
                device_id_type=pl.DeviceIdType.MESH,
            )
            rdma.start()
            return rdma

        pending = {}
        drains = []
        if _ABLATE == "barrier":
            out_ref[...] = x_ref[0, :, pl.ds(0, NP)]
            return
        for c in range(C):
            pending[(0, c)] = z_send(0, c)

        for s in range(Z - 1):
            recv_j = (mz + Z - 2 - s) % Z
            for c in range(C):
                pending[(s, c)].wait_recv()
                comm_ref[s, pl.ds(c * SH, SH)] = comm_ref[
                    s, pl.ds(c * SH, SH)
                ] + x_ref[0, pl.ds(ro + c * SH, SH), pl.ds(recv_j * NP, NP)]
                if s < Z - 2:
                    pending[(s + 1, c)] = z_send(s + 1, c)
                else:
                    final = comm_ref.at[s, pl.ds(c * SH, SH)]
                    if _ABLATE != "zonly":
                        if c == 0:
                            pl.semaphore_wait(fbar, 3)
                        for k, target in enumerate(fan):
                            drains.append(
                                swap_out(final, ro + c * SH, fsend.at[k, c],
                                         frecv.at[k, c], target)
                            )
                    out_ref[pl.ds(ro + c * SH, SH), :] = comm_ref[
                        s, pl.ds(c * SH, SH)
                    ]

        for rdma in list(pending.values()) + drains:
            rdma.wait_send()
        for rdma in drains:
            rdma.wait_recv()

    return pl.pallas_call(
        body,
        out_shape=jax.ShapeDtypeStruct((M, NP), jnp.float32),
        in_specs=[pl.BlockSpec(memory_space=pltpu.VMEM)],
        out_specs=pl.BlockSpec(memory_space=pltpu.VMEM),
        scratch_shapes=[
            pltpu.VMEM((Z - 1, MQ, NP), jnp.float32),
            pltpu.SemaphoreType.DMA((Z - 1, C)),
            pltpu.SemaphoreType.DMA((Z - 1, C)),
            pltpu.SemaphoreType.DMA((3, C)),
            pltpu.SemaphoreType.DMA((3, C)),
            pltpu.SemaphoreType.REGULAR,
        ],
        compiler_params=(
            pltpu.CompilerParams()
            if _ABLATE == "launch"
            else pltpu.CompilerParams(collective_id=0)
        ),
    )(x)


# device time: 34947 ns/iter; 1.1092x vs baseline; 1.0801x over previous
import os

import jax
import jax.numpy as jnp
from jax import lax
from jax.experimental import pallas as pl
from jax.experimental.pallas import tpu as pltpu

Z = 4
M = 1024
MQ = M // 4
C = 4
SH = MQ // C
N = 2048
NP = N // Z

_ABLATE = os.environ.get("KERNEL_ABLATE", "")


def kernel(x):
    def body(x_ref, out_ref, comm_ref, zsend, zrecv, fsend, frecv, fbar):
        mx = lax.axis_index("x")
        my = lax.axis_index("y")
        mz = lax.axis_index("z")
        left = (mz + Z - 1) % Z
        right = (mz + 1) % Z
        q = my % 2
        ypart = my - q + (1 - q)
        ro = q * 512 + mx * MQ
        rx = q * 512 + (1 - mx) * MQ
        ry = (1 - q) * 512 + mx * MQ
        fan = (
            (1 - mx, my, mz),
            (mx, ypart, mz),
        )

        if _ABLATE == "launch":
            out_ref[...] = x_ref[0, :, pl.ds(0, NP)]
            return

        barrier_sem = pltpu.get_barrier_semaphore()
        pl.semaphore_signal(
            barrier_sem, inc=1,
            device_id=(mx, my, left), device_id_type=pl.DeviceIdType.MESH,
        )
        for did in fan:
            pl.semaphore_signal(
                fbar, inc=1,
                device_id=did, device_id_type=pl.DeviceIdType.MESH,
            )
        pl.semaphore_wait(barrier_sem, 1)
        NFB = len(fan)

        def z_send(s, c):
            send_j = (mz + Z - 1 - s) % Z
            if s == 0:
                src = x_ref.at[
                    0, pl.ds(ro + c * SH, SH), pl.ds(send_j * NP, NP)
                ]
            else:
                src = comm_ref.at[s - 1, pl.ds(c * SH, SH)]
            rdma = pltpu.make_async_remote_copy(
                src_ref=src,
                dst_ref=comm_ref.at[s, pl.ds(c * SH, SH)],
                send_sem=zsend.at[s, c],
                recv_sem=zrecv.at[s, c],
                device_id=(mx, my, right),
                device_id_type=pl.DeviceIdType.MESH,
            )
            rdma.start()
            return rdma

        def swap_out(src_ref, row0, send_sem, recv_sem, target):
            rdma = pltpu.make_async_remote_copy(
                src_ref=src_ref,
                dst_ref=out_ref.at[pl.ds(row0, SH), :],
                send_sem=send_sem,
                recv_sem=recv_sem,
                device_id=target,
                device_id_type=pl.DeviceIdType.MESH,
            )
            rdma.start()
            return rdma

        pending = {}
        fanout = {}
        if _ABLATE == "barrier":
            out_ref[...] = x_ref[0, :, pl.ds(0, NP)]
            return
        for c in range(C):
            pending[(0, c)] = z_send(0, c)

        for s in range(Z - 1):
            recv_j = (mz + Z - 2 - s) % Z
            for c in range(C):
                pending[(s, c)].wait_recv()
                comm_ref[s, pl.ds(c * SH, SH)] = comm_ref[
                    s, pl.ds(c * SH, SH)
                ] + x_ref[0, pl.ds(ro + c * SH, SH), pl.ds(recv_j * NP, NP)]
                if s < Z - 2:
                    pending[(s + 1, c)] = z_send(s + 1, c)
                else:
                    final = comm_ref.at[s, pl.ds(c * SH, SH)]
                    if _ABLATE != "zonly":
                        if c == 0:
                            pl.semaphore_wait(fbar, NFB)
                        for k, target in enumerate(fan):
                            fanout[(k, c)] = swap_out(
                                final, ro + c * SH, fsend.at[k, c],
                                frecv.at[k, c], target,
                            )
                    out_ref[pl.ds(ro + c * SH, SH), :] = comm_ref[
                        s, pl.ds(c * SH, SH)
                    ]

        relays = []
        if fanout:
            for c in range(C):
                src_k = 0 if c < C // 2 else 1
                row0 = (rx if src_k == 0 else ry) + c * SH
                tgt = fan[1 - src_k]
                fanout[(src_k, c)].wait_recv()
                relays.append(
                    swap_out(
                        out_ref.at[pl.ds(row0, SH), :], row0,
                        fsend.at[2, c], frecv.at[2, c], tgt,
                    )
                )

        for rdma in list(pending.values()) + list(fanout.values()) + relays:
            rdma.wait_send()
        if fanout:
            for c in range(C):
                fanout[(1 if c < C // 2 else 0, c)].wait_recv()
            for rdma in relays:
                rdma.wait_recv()

    return pl.pallas_call(
        body,
        out_shape=jax.ShapeDtypeStruct((M, NP), jnp.float32),
        in_specs=[pl.BlockSpec(memory_space=pltpu.VMEM)],
        out_specs=pl.BlockSpec(memory_space=pltpu.VMEM),
        scratch_shapes=[
            pltpu.VMEM((Z - 1, MQ, NP), jnp.float32),
            pltpu.SemaphoreType.DMA((Z - 1, C)),
            pltpu.SemaphoreType.DMA((Z - 1, C)),
            pltpu.SemaphoreType.DMA((3, C)),
            pltpu.SemaphoreType.DMA((3, C)),
            pltpu.SemaphoreType.REGULAR,
        ],
        compiler_params=(
            pltpu.CompilerParams()
            if _ABLATE == "launch"
            else pltpu.CompilerParams(collective_id=0)
        ),
    )(x)
